# baseline (speedup 1.0000x reference)
import jax
import jax.numpy as jnp
from jax import lax
from jax.experimental import pallas as pl
from jax.experimental.pallas import tpu as pltpu
from jax.experimental.pallas import tpu_sc as plsc

N_NODES = 10000
N_EDGES = 320000
IN_SIZE = 128
HIDDEN = 256
OUT_SIZE = 128
NUM_PROP_LAYERS = 10
ALPHA = 0.1

D = OUT_SIZE
NUM_TILES = 16
EDGES_PER_TILE = N_EDGES // NUM_TILES
G = 80
RB = 80
N_ROW_CHUNKS = N_NODES // RB


def _mlp_body(x_ref, w1_ref, b1_ref, w2_ref, b2_ref, h_ref, ah_ref):
    h = jnp.maximum(
        jax.lax.dot(x_ref[...], w1_ref[...],
                    preferred_element_type=jnp.float32,
                    precision=jax.lax.Precision.HIGHEST) + b1_ref[...],
        0.0,
    )
    o = jax.lax.dot(h, w2_ref[...],
                    preferred_element_type=jnp.float32,
                    precision=jax.lax.Precision.HIGHEST) + b2_ref[...]
    h_ref[...] = o
    ah_ref[...] = o * ALPHA


def _mlp(H, W1, b1, W2, b2):
    BM = 2000
    grid = (N_NODES // BM,)
    return pl.pallas_call(
        _mlp_body,
        grid=grid,
        in_specs=[
            pl.BlockSpec((BM, IN_SIZE), lambda i: (i, 0)),
            pl.BlockSpec((IN_SIZE, HIDDEN), lambda i: (0, 0)),
            pl.BlockSpec((1, HIDDEN), lambda i: (0, 0)),
            pl.BlockSpec((HIDDEN, OUT_SIZE), lambda i: (0, 0)),
            pl.BlockSpec((1, OUT_SIZE), lambda i: (0, 0)),
        ],
        out_specs=[
            pl.BlockSpec((BM, OUT_SIZE), lambda i: (i, 0)),
            pl.BlockSpec((BM, OUT_SIZE), lambda i: (i, 0)),
        ],
        out_shape=[
            jax.ShapeDtypeStruct((N_NODES, OUT_SIZE), jnp.float32),
            jax.ShapeDtypeStruct((N_NODES, OUT_SIZE), jnp.float32),
        ],
    )(H, W1, b1.reshape(1, HIDDEN), W2, b2.reshape(1, OUT_SIZE))


NUM_WORKERS = 32
E_T = N_EDGES // NUM_WORKERS
MACRO = 2000
CHUNKS_PER_MACRO = MACRO // G
MACROS_PER_TILE = E_T // MACRO


def _scale_rows(rows, avalB, base_e):
    def scale_group(b, c3):
        av16 = avalB[pl.ds(base_e + b * 16, 16)]
        for r16 in range(16):
            sc = lax.gather(
                av16,
                jnp.full((16, 1), r16, jnp.int32),
                lax.GatherDimensionNumbers(
                    offset_dims=(),
                    collapsed_slice_dims=(0,),
                    start_index_map=(0,)),
                (1,),
                mode=lax.GatherScatterMode.PROMISE_IN_BOUNDS)
            r = b * 16 + r16
            for i in range(D // 16):
                sl = pl.ds(i * 16, 16)
                rows[r, sl] = rows[r, sl] * sc
        return c3
    lax.fori_loop(0, G // 16, scale_group, 0)


def _prop_body(hc, ah, src, dst4, aval,
               np0, np1,
               srcB, avalB, dstB, rows0, rows1, rows2,
               gsem0, gsem1, gsem2, ssem0, ssem1, ssem2, stsem, acc):
    core = lax.axis_index("c")
    sid = lax.axis_index("s")
    gw = core * NUM_TILES + sid
    tile_e0 = gw * E_T
    bufs = (rows0, rows1, rows2)
    gsems = (gsem0, gsem1, gsem2)
    ssems = (ssem0, ssem1, ssem2)

    def my_chunk(k):
        return (sid + k * NUM_TILES) * RB

    nck = (N_ROW_CHUNKS - 1 - sid) // NUM_TILES + 1

    R_T = 624
    tile_r0 = sid * R_T

    def seed(seed_ah):
        if seed_ah:
            pltpu.sync_copy(ah.at[pl.ds(tile_r0, R_T)],
                            acc.at[pl.ds(tile_r0, R_T)])

            @pl.when(sid == NUM_TILES - 1)
            def _():
                pltpu.sync_copy(ah.at[pl.ds(15 * R_T + R_T, 16)],
                                acc.at[pl.ds(15 * R_T + R_T, 16)])
        else:
            def zrow(r, c):
                for i in range(D // 16):
                    rows2[r, pl.ds(i * 16, 16)] = jnp.zeros((16,),
                                                            jnp.float32)
                return c
            lax.fori_loop(0, G, zrow, 0)

            def seed_chunk(k, carry):
                r0 = my_chunk(k)
                pltpu.sync_copy(rows2, acc.at[pl.ds(r0, RB)])
                return carry
            lax.fori_loop(0, nck, seed_chunk, 0)

    def gather_start(c, b):
        pltpu.async_copy(hc.at[srcB.at[pl.ds(c * G, G)]], bufs[b],
                         gsems[b])

    def gather_wait(c, b):
        pltpu.make_async_copy(hc.at[srcB.at[pl.ds(c * G, G)]], bufs[b],
                              gsems[b]).wait()

    def scatter_start(c, b):
        pltpu.async_copy(bufs[b], acc.at[dstB.at[c]], ssems[b], add=True)

    def scatter_wait(c, b):
        pltpu.make_async_copy(bufs[b], acc.at[dstB.at[c]], ssems[b]).wait()

    def phase_a():
        NC = CHUNKS_PER_MACRO

        def macro(m, c1):
            e0 = tile_e0 + m * MACRO
            pltpu.async_copy(src.at[pl.ds(e0, MACRO)], srcB, stsem)
            pltpu.async_copy(aval.at[pl.ds(e0, MACRO)], avalB, stsem)
            pltpu.async_copy(dst4.at[gw, m], dstB, stsem)
            pltpu.make_async_copy(src.at[pl.ds(e0, MACRO)], srcB,
                                  stsem).wait()
            gather_start(0, 0)
            gather_start(1, 1)
            pltpu.make_async_copy(aval.at[pl.ds(e0, MACRO)], avalB,
                                  stsem).wait()
            pltpu.make_async_copy(dst4.at[gw, m], dstB, stsem).wait()

            def group(g, c2):
                for b in range(3):
                    c = 3 * g + b
                    gather_wait(c, b)

                    @pl.when(c >= 1)
                    def _():
                        scatter_wait(c - 1, (b + 2) % 3)

                    @pl.when(c <= NC - 3)
                    def _():
                        gather_start(c + 2, (b + 2) % 3)

                    _scale_rows(bufs[b], avalB, c * G)
                    scatter_start(c, b)
                return c2
            lax.fori_loop(0, NC // 3, group, 0)

            last = NC - 1
            gather_wait(last, last % 3)
            _scale_rows(bufs[last % 3], avalB, last * G)
            scatter_start(last, last % 3)
            scatter_wait(last - 1, (last + 2) % 3)
            scatter_wait(last, last % 3)
            return c1
        lax.fori_loop(0, MACROS_PER_TILE, macro, 0)

    def flush(np_c):
        pltpu.sync_copy(acc.at[pl.ds(tile_r0, R_T)],
                        np_c.at[pl.ds(tile_r0, R_T)])

        @pl.when(sid == NUM_TILES - 1)
        def _():
            pltpu.sync_copy(acc.at[pl.ds(15 * R_T + R_T, 16)],
                            np_c.at[pl.ds(15 * R_T + R_T, 16)])

    @pl.when(core == 0)
    def _():
        seed(True)
        plsc.subcore_barrier()
        phase_a()
        plsc.subcore_barrier()
        flush(np0)

    @pl.when(core == 1)
    def _():
        seed(False)
        plsc.subcore_barrier()
        phase_a()
        plsc.subcore_barrier()
        flush(np1)


def _prop_step(hc, ah, src, dst4, aval):
    mesh = plsc.VectorSubcoreMesh(
        core_axis_name="c", subcore_axis_name="s")
    shp = jax.ShapeDtypeStruct((N_NODES, D), jnp.float32)
    f = pl.kernel(
        _prop_body,
        out_type=(shp, shp),
        mesh=mesh,
        scratch_types=[
            pltpu.VMEM((MACRO,), jnp.int32),
            pltpu.VMEM((MACRO,), jnp.float32),
            pltpu.VMEM((CHUNKS_PER_MACRO, G), jnp.int32),
            pltpu.VMEM((G, D), jnp.float32),
            pltpu.VMEM((G, D), jnp.float32),
            pltpu.VMEM((G, D), jnp.float32),
            pltpu.SemaphoreType.DMA,
            pltpu.SemaphoreType.DMA,
            pltpu.SemaphoreType.DMA,
            pltpu.SemaphoreType.DMA,
            pltpu.SemaphoreType.DMA,
            pltpu.SemaphoreType.DMA,
            pltpu.SemaphoreType.DMA,
            pltpu.VMEM_SHARED((N_NODES, D), jnp.float32),
        ],
    )
    return f(hc, ah, src, dst4, aval)


def _add_body(a_ref, b_ref, o_ref):
    o_ref[...] = a_ref[...] + b_ref[...]


def _tc_add(a, b):
    BM = 2000
    return pl.pallas_call(
        _add_body,
        grid=(N_NODES // BM,),
        in_specs=[pl.BlockSpec((BM, D), lambda i: (i, 0)),
                  pl.BlockSpec((BM, D), lambda i: (i, 0))],
        out_specs=pl.BlockSpec((BM, D), lambda i: (i, 0)),
        out_shape=jax.ShapeDtypeStruct((N_NODES, D), jnp.float32),
    )(a, b)


def kernel(H, A_val, edge_index, W1, b1, W2, b2):
    h_local, alpha_h = _mlp(H, W1, b1, W2, b2)
    src = edge_index[0].astype(jnp.int32)
    dst = edge_index[1].astype(jnp.int32)
    dst4 = dst.reshape(NUM_WORKERS, MACROS_PER_TILE, CHUNKS_PER_MACRO, G)
    hc = h_local
    for _ in range(NUM_PROP_LAYERS):
        p0, p1 = _prop_step(hc, alpha_h, src, dst4, A_val)
        hc = _tc_add(p0, p1)
    return hc
